# baseline (speedup 1.0000x reference)
import numpy as np
import jax
import jax.numpy as jnp
from jax.experimental import pallas as pl
from jax.experimental.pallas import tpu as pltpu

_ISIZE = 224
_PSIZE = 30
_rng = np.random.RandomState(0)
_X = int(_rng.randint(0, _ISIZE - _PSIZE))
_Y = int(_rng.randint(0, _ISIZE - _PSIZE))


def _canvas_kernel(p_ref, c_ref):
    c_ref[...] = jnp.zeros_like(c_ref)
    c_ref[:, :, _X:_X + _PSIZE, _Y:_Y + _PSIZE] = p_ref[...]


def _add_kernel(x_ref, c_ref, o_ref):
    o_ref[...] = x_ref[...] + c_ref[...]


def kernel(x, patch):
    B = x.shape[0]
    canvas = pl.pallas_call(
        _canvas_kernel,
        out_shape=jax.ShapeDtypeStruct((1, 3, _ISIZE, _ISIZE), x.dtype),
    )(patch)
    flat = 3 * _ISIZE * _ISIZE
    rows = flat // 128
    x2 = x.reshape(B, rows, 128)
    c2 = canvas.reshape(1, rows, 128)
    Bb = 8
    out = pl.pallas_call(
        _add_kernel,
        grid=(B // Bb,),
        in_specs=[
            pl.BlockSpec((Bb, rows, 128), lambda i: (i, 0, 0)),
            pl.BlockSpec((1, rows, 128), lambda i: (0, 0, 0)),
        ],
        out_specs=pl.BlockSpec((Bb, rows, 128), lambda i: (i, 0, 0)),
        out_shape=jax.ShapeDtypeStruct((B, rows, 128), x.dtype),
        compiler_params=pltpu.CompilerParams(
            dimension_semantics=("parallel",)),
    )(x2, c2)
    return out.reshape(x.shape)


# speedup vs baseline: 1.0024x; 1.0024x over previous
import numpy as np
import jax
import jax.numpy as jnp
from jax.experimental import pallas as pl
from jax.experimental.pallas import tpu as pltpu

_ISIZE = 224
_PSIZE = 30
_rng = np.random.RandomState(0)
_X = int(_rng.randint(0, _ISIZE - _PSIZE))
_Y = int(_rng.randint(0, _ISIZE - _PSIZE))

_ROWS = 3 * _ISIZE * _ISIZE // 128
_CB = 4
_R = 6


def _canvas_kernel(p_ref, c_ref):
    c_ref[...] = jnp.zeros_like(c_ref)
    c_ref[:, :, _X:_X + _PSIZE, _Y:_Y + _PSIZE] = p_ref[...]


def _add_kernel(x_hbm, c_ref, o_hbm, in_bufs, out_bufs, in_sems, out_sems):
    n_chunks = x_hbm.shape[0] // _CB

    def in_copy(c, b):
        return pltpu.make_async_copy(
            x_hbm.at[pl.ds(c * _CB, _CB)], in_bufs.at[b], in_sems.at[b])

    def out_copy(c, b):
        return pltpu.make_async_copy(
            out_bufs.at[b], o_hbm.at[pl.ds(c * _CB, _CB)], out_sems.at[b])

    for c in range(min(_R, n_chunks)):
        in_copy(c, c % _R).start()
    for c in range(n_chunks):
        b = c % _R
        in_copy(c, b).wait()
        if c >= _R:
            out_copy(c - _R, b).wait()
        out_bufs[b] = in_bufs[b] + c_ref[...]
        out_copy(c, b).start()
        if c + _R < n_chunks:
            in_copy(c + _R, b).start()
    for c in range(max(0, n_chunks - _R), n_chunks):
        out_copy(c, c % _R).wait()


def kernel(x, patch):
    B = x.shape[0]
    canvas = pl.pallas_call(
        _canvas_kernel,
        out_shape=jax.ShapeDtypeStruct((1, 3, _ISIZE, _ISIZE), x.dtype),
    )(patch)
    x2 = x.reshape(B, _ROWS, 128)
    c2 = canvas.reshape(1, _ROWS, 128)
    out = pl.pallas_call(
        _add_kernel,
        in_specs=[
            pl.BlockSpec(memory_space=pltpu.HBM),
            pl.BlockSpec(memory_space=pltpu.VMEM),
        ],
        out_specs=pl.BlockSpec(memory_space=pltpu.HBM),
        out_shape=jax.ShapeDtypeStruct((B, _ROWS, 128), x.dtype),
        scratch_shapes=[
            pltpu.VMEM((_R, _CB, _ROWS, 128), x.dtype),
            pltpu.VMEM((_R, _CB, _ROWS, 128), x.dtype),
            pltpu.SemaphoreType.DMA((_R,)),
            pltpu.SemaphoreType.DMA((_R,)),
        ],
        compiler_params=pltpu.CompilerParams(
            vmem_limit_bytes=100 * 1024 * 1024),
    )(x2, c2)
    return out.reshape(x.shape)
